# baseline (device time: 61490 ns/iter reference)
import jax
import jax.numpy as jnp
from jax import lax
from jax.experimental import pallas as pl
from jax.experimental.pallas import tpu as pltpu

NDEV = 32
NBUF = 3


def kernel(x, w_mat):
    K, mblk = x.shape
    _, N = w_mat.shape
    assert K == NDEV * mblk

    def body(x_ref, w_ref, out_ref, xsend, xrow, wtile, amax_box, amax_buf,
             send_sems, recv_sems, wsems, ag_ssems, ag_rsems):
        me = lax.axis_index("i")

        bar = pltpu.get_barrier_semaphore()
        for off in range(1, NDEV):
            dst = lax.rem(me + off, NDEV)
            pl.semaphore_signal(bar, inc=1, device_id=(dst,),
                                device_id_type=pl.DeviceIdType.MESH)
        pl.semaphore_wait(bar, NDEV - 1)

        xsend[...] = x_ref[...].astype(jnp.bfloat16)
        sends = []
        for off in range(1, NDEV):
            dst = lax.rem(me + off, NDEV)
            rdma = pltpu.make_async_remote_copy(
                src_ref=xsend.at[pl.ds(dst * mblk, mblk), :],
                dst_ref=xrow.at[me],
                send_sem=send_sems.at[off],
                recv_sem=recv_sems.at[me],
                device_id=(dst,),
                device_id_type=pl.DeviceIdType.MESH,
            )
            rdma.start()
            sends.append(rdma)

        def wcopy(j, slot):
            return pltpu.make_async_copy(
                w_ref.at[pl.ds(j * mblk, mblk), :], wtile.at[slot],
                wsems.at[slot])

        wcopy(me, 0).start()
        wcopy(lax.rem(me + NDEV - 1, NDEV), 1).start()

        for off in range(NDEV):
            slot = off % NBUF
            j = me if off == 0 else lax.rem(me - off + NDEV, NDEV)
            if off + 2 < NDEV:
                wcopy(lax.rem(me - off - 2 + NDEV, NDEV),
                      (off + 2) % NBUF).start()
            wcopy(j, slot).wait()
            if off == 0:
                xs = xsend[pl.ds(me * mblk, mblk), :]
            else:
                recv = pltpu.make_async_remote_copy(
                    src_ref=xrow.at[j], dst_ref=xrow.at[j],
                    send_sem=send_sems.at[0], recv_sem=recv_sems.at[j],
                    device_id=(me,), device_id_type=pl.DeviceIdType.MESH)
                recv.wait_recv()
                xs = xrow[j]
            wt = wtile[slot].astype(jnp.bfloat16)
            contrib = jnp.dot(xs, wt, preferred_element_type=jnp.float32)
            if off == 0:
                out_ref[...] = contrib
            else:
                out_ref[...] += contrib

        local_amax = jnp.max(jnp.abs(out_ref[...]))
        amax_box[...] = jnp.full((8, 128), local_amax, jnp.float32)
        ag_sends = []
        for off in range(1, NDEV):
            dst = lax.rem(me + off, NDEV)
            rdma = pltpu.make_async_remote_copy(
                src_ref=amax_box,
                dst_ref=amax_buf.at[me],
                send_sem=ag_ssems.at[off],
                recv_sem=ag_rsems.at[me],
                device_id=(dst,),
                device_id_type=pl.DeviceIdType.MESH,
            )
            rdma.start()
            ag_sends.append(rdma)
        for off in range(1, NDEV):
            j = lax.rem(me + off, NDEV)
            recv = pltpu.make_async_remote_copy(
                src_ref=amax_box, dst_ref=amax_buf.at[j],
                send_sem=ag_ssems.at[0], recv_sem=ag_rsems.at[j],
                device_id=(me,), device_id_type=pl.DeviceIdType.MESH)
            recv.wait_recv()

        rows = lax.broadcasted_iota(jnp.int32, (NDEV, 8, 128), 0)
        others = jnp.where(rows == me, -jnp.inf, amax_buf[...])
        g = jnp.maximum(jnp.max(others), local_amax)

        scale = g / 448.0
        r = jnp.clip(out_ref[...] * (448.0 / g), -448.0, 448.0)
        u = lax.bitcast_convert_type(r, jnp.int32)
        u = (u + 0x7FFFF + ((u >> 20) & 1)) & ~0xFFFFF
        q = lax.bitcast_convert_type(u, jnp.float32)
        out_ref[...] = q * scale

        for rdma in sends:
            rdma.wait_send()
        for rdma in ag_sends:
            rdma.wait_send()

    return pl.pallas_call(
        body,
        out_shape=jax.ShapeDtypeStruct((mblk, N), jnp.float32),
        in_specs=[
            pl.BlockSpec(memory_space=pltpu.MemorySpace.VMEM),
            pl.BlockSpec(memory_space=pltpu.MemorySpace.HBM),
        ],
        out_specs=pl.BlockSpec(memory_space=pltpu.MemorySpace.VMEM),
        scratch_shapes=[
            pltpu.VMEM((K, mblk), jnp.bfloat16),
            pltpu.VMEM((NDEV, mblk, mblk), jnp.bfloat16),
            pltpu.VMEM((NBUF, mblk, N), jnp.float32),
            pltpu.VMEM((8, 128), jnp.float32),
            pltpu.VMEM((NDEV, 8, 128), jnp.float32),
            pltpu.SemaphoreType.DMA((NDEV,)),
            pltpu.SemaphoreType.DMA((NDEV,)),
            pltpu.SemaphoreType.DMA((NBUF,)),
            pltpu.SemaphoreType.DMA((NDEV,)),
            pltpu.SemaphoreType.DMA((NDEV,)),
        ],
        compiler_params=pltpu.CompilerParams(collective_id=0),
    )(x, w_mat)


# device time: 60656 ns/iter; 1.0137x vs baseline; 1.0137x over previous
import jax
import jax.numpy as jnp
from jax import lax
from jax.experimental import pallas as pl
from jax.experimental.pallas import tpu as pltpu

NDEV = 32
NBUF = 4
NSPLIT = 2


def kernel(x, w_mat):
    K, mblk = x.shape
    _, N = w_mat.shape
    assert K == NDEV * mblk

    def body(x_ref, w_ref, out_ref, xsend, xrow, wtile, amax_box, amax_buf,
             send_sems, recv_sems, wsems, ag_ssems, ag_rsems):
        me = lax.axis_index("i")

        bar = pltpu.get_barrier_semaphore()
        for off in range(1, NDEV):
            dst = lax.rem(me + off, NDEV)
            pl.semaphore_signal(bar, inc=1, device_id=(dst,),
                                device_id_type=pl.DeviceIdType.MESH)
        pl.semaphore_wait(bar, NDEV - 1)

        xsend[...] = x_ref[...].astype(jnp.bfloat16)
        sends = []
        for off in range(1, NDEV):
            dst = lax.rem(me + off, NDEV)
            rdma = pltpu.make_async_remote_copy(
                src_ref=xsend.at[pl.ds(dst * mblk, mblk), :],
                dst_ref=xrow.at[me],
                send_sem=send_sems.at[off],
                recv_sem=recv_sems.at[me],
                device_id=(dst,),
                device_id_type=pl.DeviceIdType.MESH,
            )
            rdma.start()
            sends.append(rdma)

        half = mblk // NSPLIT

        def wcopies(j, slot):
            return [
                pltpu.make_async_copy(
                    w_ref.at[pl.ds(j * mblk + s * half, half), :],
                    wtile.at[slot, pl.ds(s * half, half), :],
                    wsems.at[slot, s])
                for s in range(NSPLIT)
            ]

        def wstart(j, slot):
            for c in wcopies(j, slot):
                c.start()

        PREFETCH = NBUF - 1
        for p in range(PREFETCH):
            wstart(me if p == 0 else lax.rem(me - p + NDEV, NDEV), p)

        for off in range(NDEV):
            slot = off % NBUF
            j = me if off == 0 else lax.rem(me - off + NDEV, NDEV)
            if off + PREFETCH < NDEV:
                wstart(lax.rem(me - off - PREFETCH + NDEV, NDEV),
                       (off + PREFETCH) % NBUF)
            for c in wcopies(j, slot):
                c.wait()
            if off == 0:
                xs = xsend[pl.ds(me * mblk, mblk), :]
            else:
                recv = pltpu.make_async_remote_copy(
                    src_ref=xrow.at[j], dst_ref=xrow.at[j],
                    send_sem=send_sems.at[0], recv_sem=recv_sems.at[j],
                    device_id=(me,), device_id_type=pl.DeviceIdType.MESH)
                recv.wait_recv()
                xs = xrow[j]
            wt = wtile[slot].astype(jnp.bfloat16)
            contrib = jnp.dot(xs, wt, preferred_element_type=jnp.float32)
            if off == 0:
                out_ref[...] = contrib
            else:
                out_ref[...] += contrib

        local_amax = jnp.max(jnp.abs(out_ref[...]))
        amax_box[...] = jnp.full((8, 128), local_amax, jnp.float32)
        ag_sends = []
        for off in range(1, NDEV):
            dst = lax.rem(me + off, NDEV)
            rdma = pltpu.make_async_remote_copy(
                src_ref=amax_box,
                dst_ref=amax_buf.at[me],
                send_sem=ag_ssems.at[off],
                recv_sem=ag_rsems.at[me],
                device_id=(dst,),
                device_id_type=pl.DeviceIdType.MESH,
            )
            rdma.start()
            ag_sends.append(rdma)
        for off in range(1, NDEV):
            j = lax.rem(me + off, NDEV)
            recv = pltpu.make_async_remote_copy(
                src_ref=amax_box, dst_ref=amax_buf.at[j],
                send_sem=ag_ssems.at[0], recv_sem=ag_rsems.at[j],
                device_id=(me,), device_id_type=pl.DeviceIdType.MESH)
            recv.wait_recv()

        rows = lax.broadcasted_iota(jnp.int32, (NDEV, 8, 128), 0)
        others = jnp.where(rows == me, -jnp.inf, amax_buf[...])
        g = jnp.maximum(jnp.max(others), local_amax)

        scale = g / 448.0
        r = jnp.clip(out_ref[...] * (448.0 / g), -448.0, 448.0)
        q = r.astype(jnp.float8_e4m3fn).astype(jnp.float32)
        out_ref[...] = q * scale

        for rdma in sends:
            rdma.wait_send()
        for rdma in ag_sends:
            rdma.wait_send()

    return pl.pallas_call(
        body,
        out_shape=jax.ShapeDtypeStruct((mblk, N), jnp.float32),
        in_specs=[
            pl.BlockSpec(memory_space=pltpu.MemorySpace.VMEM),
            pl.BlockSpec(memory_space=pltpu.MemorySpace.HBM),
        ],
        out_specs=pl.BlockSpec(memory_space=pltpu.MemorySpace.VMEM),
        scratch_shapes=[
            pltpu.VMEM((K, mblk), jnp.bfloat16),
            pltpu.VMEM((NDEV, mblk, mblk), jnp.bfloat16),
            pltpu.VMEM((NBUF, mblk, N), jnp.float32),
            pltpu.VMEM((8, 128), jnp.float32),
            pltpu.VMEM((NDEV, 8, 128), jnp.float32),
            pltpu.SemaphoreType.DMA((NDEV,)),
            pltpu.SemaphoreType.DMA((NDEV,)),
            pltpu.SemaphoreType.DMA((NBUF, NSPLIT)),
            pltpu.SemaphoreType.DMA((NDEV,)),
            pltpu.SemaphoreType.DMA((NDEV,)),
        ],
        compiler_params=pltpu.CompilerParams(collective_id=0),
    )(x, w_mat)


# device time: 53209 ns/iter; 1.1556x vs baseline; 1.1400x over previous
import os

import jax
import jax.numpy as jnp
from jax import lax
from jax.experimental import pallas as pl
from jax.experimental.pallas import tpu as pltpu

ABLATE = os.environ.get("KERNEL_ABLATE", "none")

NDEV = 32
NBUF = 4
NSPLIT = 2


def kernel(x, w_mat):
    K, mblk = x.shape
    _, N = w_mat.shape
    assert K == NDEV * mblk

    def body(x_ref, w_ref, out_ref, xsend, xrow, wtile, amax_box, amax_buf,
             send_sems, recv_sems, wsems, ag_ssems, ag_rsems):
        me = lax.axis_index("i")

        bar = pltpu.get_barrier_semaphore()
        for off in range(1, NDEV):
            dst = lax.rem(me + off, NDEV)
            pl.semaphore_signal(bar, inc=1, device_id=(dst,),
                                device_id_type=pl.DeviceIdType.MESH)
        pl.semaphore_wait(bar, NDEV - 1)

        use_comm = ABLATE not in ("gemmonly", "base", "wonly")
        use_w = ABLATE not in ("noepi_now", "noepi_nogemm", "base")
        use_dot = ABLATE not in ("noepi_nogemm", "base", "wonly")

        xsend[...] = x_ref[...].astype(jnp.bfloat16)
        sends = []
        for off in range(1, NDEV) if use_comm else []:
            dst = lax.rem(me + off, NDEV)
            rdma = pltpu.make_async_remote_copy(
                src_ref=xsend.at[pl.ds(dst * mblk, mblk), :],
                dst_ref=xrow.at[me],
                send_sem=send_sems.at[off],
                recv_sem=recv_sems.at[me],
                device_id=(dst,),
                device_id_type=pl.DeviceIdType.MESH,
            )
            rdma.start()
            sends.append(rdma)

        half = mblk // NSPLIT

        def wcopies(j, slot):
            return [
                pltpu.make_async_copy(
                    w_ref.at[pl.ds(j * mblk + s * half, half), :],
                    wtile.at[slot, pl.ds(s * half, half), :],
                    wsems.at[slot, s])
                for s in range(NSPLIT)
            ]

        def wstart(j, slot):
            for c in wcopies(j, slot):
                c.start()

        PREFETCH = NBUF - 1
        if use_w:
            for p in range(PREFETCH):
                wstart(me if p == 0 else lax.rem(me - p + NDEV, NDEV), p)

        for off in range(NDEV):
            slot = off % NBUF
            j = me if off == 0 else lax.rem(me - off + NDEV, NDEV)
            if use_w:
                if off + PREFETCH < NDEV:
                    wstart(lax.rem(me - off - PREFETCH + NDEV, NDEV),
                           (off + PREFETCH) % NBUF)
                for c in wcopies(j, slot):
                    c.wait()
            if off == 0 or not use_comm:
                xs = xsend[pl.ds(me * mblk, mblk), :]
            else:
                recv = pltpu.make_async_remote_copy(
                    src_ref=xrow.at[j], dst_ref=xrow.at[j],
                    send_sem=send_sems.at[0], recv_sem=recv_sems.at[j],
                    device_id=(me,), device_id_type=pl.DeviceIdType.MESH)
                recv.wait_recv()
                xs = xrow[j]
            if use_dot:
                wt = wtile[slot].astype(jnp.bfloat16)
                contrib = jnp.dot(xs, wt, preferred_element_type=jnp.float32)
                if off == 0:
                    out_ref[...] = contrib
                else:
                    out_ref[...] += contrib
        if not use_dot:
            out_ref[...] = jnp.full((mblk, N), 1.0, jnp.float32)

        do_epi = ABLATE == "none"
        local_amax = jnp.max(jnp.abs(out_ref[...]))
        ag_sends = []
        if do_epi:
            amax_box[...] = jnp.full((8, 128), local_amax, jnp.float32)
            for off in range(1, NDEV):
                dst = lax.rem(me + off, NDEV)
                rdma = pltpu.make_async_remote_copy(
                    src_ref=amax_box,
                    dst_ref=amax_buf.at[me],
                    send_sem=ag_ssems.at[off],
                    recv_sem=ag_rsems.at[me],
                    device_id=(dst,),
                    device_id_type=pl.DeviceIdType.MESH,
                )
                rdma.start()
                ag_sends.append(rdma)
            for off in range(1, NDEV):
                j = lax.rem(me + off, NDEV)
                recv = pltpu.make_async_remote_copy(
                    src_ref=amax_box, dst_ref=amax_buf.at[j],
                    send_sem=ag_ssems.at[0], recv_sem=ag_rsems.at[j],
                    device_id=(me,), device_id_type=pl.DeviceIdType.MESH)
                recv.wait_recv()

            rows = lax.broadcasted_iota(jnp.int32, (NDEV, 8, 128), 0)
            others = jnp.where(rows == me, -jnp.inf, amax_buf[...])
            g = jnp.maximum(jnp.max(others), local_amax)

            scale = g / 448.0
            r = jnp.clip(out_ref[...] * (448.0 / g), -448.0, 448.0)
            q = r.astype(jnp.float8_e4m3fn).astype(jnp.float32)
            out_ref[...] = q * scale

        for rdma in sends:
            rdma.wait_send()
        for rdma in ag_sends:
            rdma.wait_send()

    return pl.pallas_call(
        body,
        out_shape=jax.ShapeDtypeStruct((mblk, N), jnp.float32),
        in_specs=[
            pl.BlockSpec(memory_space=pltpu.MemorySpace.VMEM),
            pl.BlockSpec(memory_space=pltpu.MemorySpace.HBM),
        ],
        out_specs=pl.BlockSpec(memory_space=pltpu.MemorySpace.VMEM),
        scratch_shapes=[
            pltpu.VMEM((K, mblk), jnp.bfloat16),
            pltpu.VMEM((NDEV, mblk, mblk), jnp.bfloat16),
            pltpu.VMEM((NBUF, mblk, N), jnp.float32),
            pltpu.VMEM((8, 128), jnp.float32),
            pltpu.VMEM((NDEV, 8, 128), jnp.float32),
            pltpu.SemaphoreType.DMA((NDEV,)),
            pltpu.SemaphoreType.DMA((NDEV,)),
            pltpu.SemaphoreType.DMA((NBUF, NSPLIT)),
            pltpu.SemaphoreType.DMA((NDEV,)),
            pltpu.SemaphoreType.DMA((NDEV,)),
        ],
        compiler_params=pltpu.CompilerParams(collective_id=0),
    )(x, w_mat)
